# baseline (speedup 1.0000x reference)
import functools

import jax
import jax.numpy as jnp
from jax import lax
from jax.experimental import pallas as pl
from jax.experimental.pallas import tpu as pltpu
from jax.experimental.pallas import tpu_sc as plsc

NUM_EXPERTS = 8
TOP_K = 2
TILE = 512
GTILE = 128
NC, NS = 2, 16
NW = NC * NS


def _top2(scores):
    eidx = lax.broadcasted_iota(jnp.int32, scores.shape, 1)
    m1 = jnp.max(scores, axis=-1, keepdims=True)
    top1 = jnp.min(jnp.where(scores == m1, eidx, NUM_EXPERTS),
                   axis=-1, keepdims=True)
    masked = jnp.where(eidx == top1, -jnp.inf, scores)
    m2 = jnp.max(masked, axis=-1, keepdims=True)
    top2 = jnp.min(jnp.where(masked == m2, eidx, NUM_EXPERTS),
                   axis=-1, keepdims=True)
    z2 = jnp.exp(m2 - m1)
    denom = 1.0 + z2
    return top1, top2, 1.0 / denom, z2 / denom


def _router_kernel(x_ref, gate_ref, r_ref):
    scores = lax.dot_general(x_ref[...], gate_ref[...], (((1,), (1,)), ((), ())),
                             preferred_element_type=jnp.float32)
    eidx = lax.broadcasted_iota(jnp.int32, scores.shape, 1)
    top1, top2, _, _ = _top2(scores)
    r_ref[...] = (jnp.where(eidx == top1, 1, 0)
                  + jnp.where(eidx == top2, 2, 0)).astype(jnp.int32)


def _grouped_kernel(te_ref, x_ref, gate_ref, w1_ref, b1_ref, w2_ref, b2_ref,
                    out_ref):
    g = pl.program_id(0)
    e = te_ref[g]
    xt = x_ref[...]

    scores = lax.dot_general(xt, gate_ref[...], (((1,), (1,)), ((), ())),
                             preferred_element_type=jnp.float32)
    top1, top2, p1, p2 = _top2(scores)
    weight = jnp.where(top1 == e, p1, 0.0) + jnp.where(top2 == e, p2, 0.0)

    h = lax.dot_general(xt, w1_ref[0], (((1,), (1,)), ((), ())),
                        preferred_element_type=jnp.float32)
    h = h + b1_ref[0]
    f = h.shape[-1] // 2
    a = h[:, :f]
    gt = h[:, f:]
    hidden = (a * jax.nn.sigmoid(a)) * gt
    eo = lax.dot_general(hidden, w2_ref[0], (((1,), (1,)), ((), ())),
                         preferred_element_type=jnp.float32)
    out_ref[...] = (eo + b2_ref[0]) * weight


def _add_kernel(a_ref, b_ref, out_ref):
    out_ref[...] = a_ref[...] + b_ref[...]


def _pick_chunk(per_w, d):
    budget = 450_000 // (3 * d * 4)
    c = per_w
    while c > budget or c % 8 != 0 or per_w % c != 0:
        c -= 8
    return c


def _make_row_gather(n_rows, n_out, d):
    per_w = n_out // NW
    chunk = _pick_chunk(per_w, d)
    n_chunks = per_w // chunk
    mesh = plsc.VectorSubcoreMesh(core_axis_name="c", subcore_axis_name="s")

    @functools.partial(
        pl.kernel, mesh=mesh,
        out_type=jax.ShapeDtypeStruct((n_out, d), jnp.float32),
        scratch_types=[
            pltpu.VMEM((3, chunk), jnp.int32),
            pltpu.VMEM((chunk, d), jnp.float32),
            pltpu.VMEM((chunk, d), jnp.float32),
            pltpu.VMEM((chunk, d), jnp.float32),
            pltpu.SemaphoreType.DMA,
            pltpu.SemaphoreType.DMA,
            pltpu.SemaphoreType.DMA,
            pltpu.SemaphoreType.DMA,
            pltpu.SemaphoreType.DMA,
            pltpu.SemaphoreType.DMA,
        ],
    )
    def gather(table_hbm, idx_hbm, out_hbm, idx_v, r0, r1, r2,
               g0, g1, g2, w0, w1, w2):
        wid = lax.axis_index("s") * NC + lax.axis_index("c")
        base = wid * per_w
        bufs = (r0, r1, r2)
        gsems = (g0, g1, g2)
        wsems = (w0, w1, w2)

        def fire_gather(c):
            b = c % 3
            pltpu.sync_copy(idx_hbm.at[pl.ds(base + c * chunk, chunk)],
                            idx_v.at[b])
            return pltpu.async_copy(table_hbm.at[idx_v.at[b]], bufs[b],
                                    gsems[b])

        gcp = [None, None, None]
        wcp = [None, None, None]
        gcp[0] = fire_gather(0)
        if n_chunks > 1:
            gcp[1] = fire_gather(1)
        for c in range(n_chunks):
            b = c % 3
            gcp[b].wait()
            wcp[b] = pltpu.async_copy(
                bufs[b], out_hbm.at[pl.ds(base + c * chunk, chunk)], wsems[b])
            nxt = c + 2
            if nxt < n_chunks:
                nb = nxt % 3
                if wcp[nb] is not None:
                    wcp[nb].wait()
                gcp[nb] = fire_gather(nxt)
        for b in range(min(3, n_chunks)):
            if wcp[b] is not None:
                wcp[b].wait()

    return gather


@jax.jit
def kernel(x, gate_w, w1, b1, w2, b2):
    bsz, seq, d = x.shape
    n = bsz * seq
    xf = x.reshape(n, d)
    two_f = w1.shape[1]
    n_assign = n * TOP_K
    n_groups = n_assign // GTILE + NUM_EXPERTS
    pad_len = n_groups * GTILE

    r = pl.pallas_call(
        _router_kernel,
        grid=(n // TILE,),
        in_specs=[
            pl.BlockSpec((TILE, d), lambda t: (t, 0)),
            pl.BlockSpec(gate_w.shape, lambda t: (0, 0)),
        ],
        out_specs=pl.BlockSpec((TILE, NUM_EXPERTS), lambda t: (t, 0)),
        out_shape=jax.ShapeDtypeStruct((n, NUM_EXPERTS), jnp.int32),
    )(xf, gate_w)

    e0 = jnp.argmax(r == 1, axis=1).astype(jnp.int32)
    e1 = jnp.argmax(r == 2, axis=1).astype(jnp.int32)
    expert_ids = jnp.concatenate([e0, e1])
    onehot = (expert_ids[:, None] == jnp.arange(NUM_EXPERTS)).astype(jnp.int32)
    incl = jnp.cumsum(onehot, axis=0)
    rank_within = jnp.sum((incl - onehot) * onehot, axis=1)
    counts = incl[-1]
    padded_counts = ((counts + GTILE - 1) // GTILE) * GTILE
    padded_offsets = jnp.concatenate(
        [jnp.zeros((1,), jnp.int32),
         jnp.cumsum(padded_counts)[:-1].astype(jnp.int32)])
    dest = padded_offsets[expert_ids] + rank_within
    tok = jnp.arange(n, dtype=jnp.int32)
    token_ids = jnp.concatenate([tok, tok])
    sorted_token = jnp.zeros((pad_len,), jnp.int32).at[dest].set(token_ids)
    tile_starts = jnp.arange(n_groups, dtype=jnp.int32) * GTILE
    tile_expert = jnp.clip(
        jnp.sum(tile_starts[:, None] >= padded_offsets[None, :], axis=1) - 1,
        0, NUM_EXPERTS - 1).astype(jnp.int32)

    xg = _make_row_gather(n, pad_len, d)(xf, sorted_token)

    yg = pl.pallas_call(
        _grouped_kernel,
        grid_spec=pltpu.PrefetchScalarGridSpec(
            num_scalar_prefetch=1,
            grid=(n_groups,),
            in_specs=[
                pl.BlockSpec((GTILE, d), lambda g, te: (g, 0)),
                pl.BlockSpec(gate_w.shape, lambda g, te: (0, 0)),
                pl.BlockSpec((1, two_f, d), lambda g, te: (te[g], 0, 0)),
                pl.BlockSpec((1, 1, two_f), lambda g, te: (te[g], 0, 0)),
                pl.BlockSpec((1, d, two_f // 2), lambda g, te: (te[g], 0, 0)),
                pl.BlockSpec((1, 1, d), lambda g, te: (te[g], 0, 0)),
            ],
            out_specs=pl.BlockSpec((GTILE, d), lambda g, te: (g, 0)),
        ),
        out_shape=jax.ShapeDtypeStruct((pad_len, d), jnp.float32),
    )(tile_expert, xg, gate_w, w1, b1.reshape(NUM_EXPERTS, 1, two_f), w2,
      b2.reshape(NUM_EXPERTS, 1, d))

    ygg = _make_row_gather(pad_len, n_assign, d)(yg, dest)

    nt = n // TILE
    out = pl.pallas_call(
        _add_kernel,
        grid=(nt,),
        in_specs=[
            pl.BlockSpec((TILE, d), lambda t: (t, 0)),
            pl.BlockSpec((TILE, d), lambda t: (t + nt, 0)),
        ],
        out_specs=pl.BlockSpec((TILE, d), lambda t: (t, 0)),
        out_shape=jax.ShapeDtypeStruct((n, d), jnp.float32),
    )(ygg, ygg)

    return out.reshape(bsz, seq, d), jnp.float32(0.0)


# speedup vs baseline: 1.0061x; 1.0061x over previous
import functools

import jax
import jax.numpy as jnp
from jax import lax
from jax.experimental import pallas as pl
from jax.experimental.pallas import tpu as pltpu
from jax.experimental.pallas import tpu_sc as plsc

NUM_EXPERTS = 8
TOP_K = 2
TILE = 512
GTILE = 128
NC, NS = 2, 16
NW = NC * NS


def _top2(scores):
    eidx = lax.broadcasted_iota(jnp.int32, scores.shape, 1)
    m1 = jnp.max(scores, axis=-1, keepdims=True)
    top1 = jnp.min(jnp.where(scores == m1, eidx, NUM_EXPERTS),
                   axis=-1, keepdims=True)
    masked = jnp.where(eidx == top1, -jnp.inf, scores)
    m2 = jnp.max(masked, axis=-1, keepdims=True)
    top2 = jnp.min(jnp.where(masked == m2, eidx, NUM_EXPERTS),
                   axis=-1, keepdims=True)
    z2 = jnp.exp(m2 - m1)
    denom = 1.0 + z2
    return top1, top2, 1.0 / denom, z2 / denom


def _router_kernel(x_ref, gate_ref, r_ref):
    scores = lax.dot_general(x_ref[...], gate_ref[...], (((1,), (1,)), ((), ())),
                             preferred_element_type=jnp.float32)
    eidx = lax.broadcasted_iota(jnp.int32, scores.shape, 1)
    top1, top2, _, _ = _top2(scores)
    r_ref[...] = (jnp.where(eidx == top1, 1, 0)
                  + jnp.where(eidx == top2, 2, 0)).astype(jnp.int32)


def _grouped_kernel(te_ref, x_ref, gate_ref, w1_ref, b1_ref, w2_ref, b2_ref,
                    out_ref):
    g = pl.program_id(0)
    e = te_ref[g]
    xt = x_ref[...]

    scores = lax.dot_general(xt, gate_ref[...], (((1,), (1,)), ((), ())),
                             preferred_element_type=jnp.float32)
    top1, top2, p1, p2 = _top2(scores)
    weight = jnp.where(top1 == e, p1, 0.0) + jnp.where(top2 == e, p2, 0.0)

    h = lax.dot_general(xt, w1_ref[0], (((1,), (1,)), ((), ())),
                        preferred_element_type=jnp.float32)
    h = h + b1_ref[0]
    f = h.shape[-1] // 2
    a = h[:, :f]
    gt = h[:, f:]
    hidden = (a * jax.nn.sigmoid(a)) * gt
    eo = lax.dot_general(hidden, w2_ref[0], (((1,), (1,)), ((), ())),
                         preferred_element_type=jnp.float32)
    out_ref[...] = (eo + b2_ref[0]) * weight


def _add_kernel(a_ref, b_ref, out_ref):
    out_ref[...] = a_ref[...] + b_ref[...]


NBUF = 6
GDEPTH = 4


def _pick_chunk(per_w, d):
    budget = 440_000 // (NBUF * d * 4)
    c = (budget // 8) * 8
    while c % 8 != 0 or per_w % c != 0:
        c -= 8
    return c


def _make_row_gather(n_rows, n_out, d):
    per_w = n_out // NW
    chunk = _pick_chunk(per_w, d)
    n_chunks = per_w // chunk
    mesh = plsc.VectorSubcoreMesh(core_axis_name="c", subcore_axis_name="s")

    @functools.partial(
        pl.kernel, mesh=mesh,
        out_type=jax.ShapeDtypeStruct((n_out, d), jnp.float32),
        scratch_types=(
            [pltpu.VMEM((per_w,), jnp.int32)]
            + [pltpu.VMEM((chunk, d), jnp.float32)] * NBUF
            + [pltpu.SemaphoreType.DMA] * (2 * NBUF)
        ),
    )
    def gather(table_hbm, idx_hbm, out_hbm, idx_v, *bufs_sems):
        bufs = bufs_sems[:NBUF]
        gsems = bufs_sems[NBUF:2 * NBUF]
        wsems = bufs_sems[2 * NBUF:]
        wid = lax.axis_index("s") * NC + lax.axis_index("c")
        base = wid * per_w

        pltpu.sync_copy(idx_hbm.at[pl.ds(base, per_w)], idx_v)

        def fire_gather(c):
            b = c % NBUF
            return pltpu.async_copy(
                table_hbm.at[idx_v.at[pl.ds(c * chunk, chunk)]], bufs[b],
                gsems[b])

        gcp = [None] * NBUF
        wcp = [None] * NBUF
        for c in range(min(GDEPTH, n_chunks)):
            gcp[c % NBUF] = fire_gather(c)
        for c in range(n_chunks):
            b = c % NBUF
            gcp[b].wait()
            wcp[b] = pltpu.async_copy(
                bufs[b], out_hbm.at[pl.ds(base + c * chunk, chunk)], wsems[b])
            nxt = c + GDEPTH
            if nxt < n_chunks:
                nb = nxt % NBUF
                if wcp[nb] is not None:
                    wcp[nb].wait()
                gcp[nb] = fire_gather(nxt)
        for b in range(min(NBUF, n_chunks)):
            if wcp[b] is not None:
                wcp[b].wait()

    return gather


@jax.jit
def kernel(x, gate_w, w1, b1, w2, b2):
    bsz, seq, d = x.shape
    n = bsz * seq
    xf = x.reshape(n, d)
    two_f = w1.shape[1]
    n_assign = n * TOP_K
    n_groups = n_assign // GTILE + NUM_EXPERTS
    pad_len = n_groups * GTILE

    r = pl.pallas_call(
        _router_kernel,
        grid=(n // TILE,),
        in_specs=[
            pl.BlockSpec((TILE, d), lambda t: (t, 0)),
            pl.BlockSpec(gate_w.shape, lambda t: (0, 0)),
        ],
        out_specs=pl.BlockSpec((TILE, NUM_EXPERTS), lambda t: (t, 0)),
        out_shape=jax.ShapeDtypeStruct((n, NUM_EXPERTS), jnp.int32),
    )(xf, gate_w)

    e0 = jnp.argmax(r == 1, axis=1).astype(jnp.int32)
    e1 = jnp.argmax(r == 2, axis=1).astype(jnp.int32)
    expert_ids = jnp.concatenate([e0, e1])
    onehot = (expert_ids[:, None] == jnp.arange(NUM_EXPERTS)).astype(jnp.int32)
    incl = jnp.cumsum(onehot, axis=0)
    rank_within = jnp.sum((incl - onehot) * onehot, axis=1)
    counts = incl[-1]
    padded_counts = ((counts + GTILE - 1) // GTILE) * GTILE
    padded_offsets = jnp.concatenate(
        [jnp.zeros((1,), jnp.int32),
         jnp.cumsum(padded_counts)[:-1].astype(jnp.int32)])
    dest = padded_offsets[expert_ids] + rank_within
    tok = jnp.arange(n, dtype=jnp.int32)
    token_ids = jnp.concatenate([tok, tok])
    sorted_token = jnp.zeros((pad_len,), jnp.int32).at[dest].set(token_ids)
    tile_starts = jnp.arange(n_groups, dtype=jnp.int32) * GTILE
    tile_expert = jnp.clip(
        jnp.sum(tile_starts[:, None] >= padded_offsets[None, :], axis=1) - 1,
        0, NUM_EXPERTS - 1).astype(jnp.int32)

    xg = _make_row_gather(n, pad_len, d)(xf, sorted_token)

    yg = pl.pallas_call(
        _grouped_kernel,
        grid_spec=pltpu.PrefetchScalarGridSpec(
            num_scalar_prefetch=1,
            grid=(n_groups,),
            in_specs=[
                pl.BlockSpec((GTILE, d), lambda g, te: (g, 0)),
                pl.BlockSpec(gate_w.shape, lambda g, te: (0, 0)),
                pl.BlockSpec((1, two_f, d), lambda g, te: (te[g], 0, 0)),
                pl.BlockSpec((1, 1, two_f), lambda g, te: (te[g], 0, 0)),
                pl.BlockSpec((1, d, two_f // 2), lambda g, te: (te[g], 0, 0)),
                pl.BlockSpec((1, 1, d), lambda g, te: (te[g], 0, 0)),
            ],
            out_specs=pl.BlockSpec((GTILE, d), lambda g, te: (g, 0)),
        ),
        out_shape=jax.ShapeDtypeStruct((pad_len, d), jnp.float32),
    )(tile_expert, xg, gate_w, w1, b1.reshape(NUM_EXPERTS, 1, two_f), w2,
      b2.reshape(NUM_EXPERTS, 1, d))

    ygg = _make_row_gather(pad_len, n_assign, d)(yg, dest)

    nt = n // TILE
    out = pl.pallas_call(
        _add_kernel,
        grid=(nt,),
        in_specs=[
            pl.BlockSpec((TILE, d), lambda t: (t, 0)),
            pl.BlockSpec((TILE, d), lambda t: (t + nt, 0)),
        ],
        out_specs=pl.BlockSpec((TILE, d), lambda t: (t, 0)),
        out_shape=jax.ShapeDtypeStruct((n, d), jnp.float32),
    )(ygg, ygg)

    return out.reshape(bsz, seq, d), jnp.float32(0.0)


# speedup vs baseline: 1.0062x; 1.0001x over previous
import functools

import jax
import jax.numpy as jnp
from jax import lax
from jax.experimental import pallas as pl
from jax.experimental.pallas import tpu as pltpu
from jax.experimental.pallas import tpu_sc as plsc

NUM_EXPERTS = 8
TOP_K = 2
TILE = 512
GTILE = 128
NC, NS = 2, 16
NW = NC * NS


def _top2(scores):
    eidx = lax.broadcasted_iota(jnp.int32, scores.shape, 1)
    m1 = jnp.max(scores, axis=-1, keepdims=True)
    top1 = jnp.min(jnp.where(scores == m1, eidx, NUM_EXPERTS),
                   axis=-1, keepdims=True)
    masked = jnp.where(eidx == top1, -jnp.inf, scores)
    m2 = jnp.max(masked, axis=-1, keepdims=True)
    top2 = jnp.min(jnp.where(masked == m2, eidx, NUM_EXPERTS),
                   axis=-1, keepdims=True)
    z2 = jnp.exp(m2 - m1)
    denom = 1.0 + z2
    return top1, top2, 1.0 / denom, z2 / denom


def _router_kernel(x_ref, gate_ref, r_ref):
    scores = lax.dot_general(x_ref[...], gate_ref[...], (((1,), (1,)), ((), ())),
                             preferred_element_type=jnp.float32)
    eidx = lax.broadcasted_iota(jnp.int32, scores.shape, 1)
    top1, top2, _, _ = _top2(scores)
    r_ref[...] = (jnp.where(eidx == top1, 1, 0)
                  + jnp.where(eidx == top2, 2, 0)).astype(jnp.int32)


def _grouped_kernel(te_ref, x_ref, gate_ref, w1_ref, b1_ref, w2_ref, b2_ref,
                    out_ref):
    g = pl.program_id(0)
    e = te_ref[g]
    xt = x_ref[...]

    scores = lax.dot_general(xt, gate_ref[...], (((1,), (1,)), ((), ())),
                             preferred_element_type=jnp.float32)
    top1, top2, p1, p2 = _top2(scores)
    weight = jnp.where(top1 == e, p1, 0.0) + jnp.where(top2 == e, p2, 0.0)

    h = lax.dot_general(xt, w1_ref[0], (((1,), (1,)), ((), ())),
                        preferred_element_type=jnp.float32)
    h = h + b1_ref[0]
    f = h.shape[-1] // 2
    a = h[:, :f]
    gt = h[:, f:]
    hidden = (a * jax.nn.sigmoid(a)) * gt
    eo = lax.dot_general(hidden, w2_ref[0], (((1,), (1,)), ((), ())),
                         preferred_element_type=jnp.float32)
    out_ref[...] = (eo + b2_ref[0]) * weight


def _add_kernel(a_ref, b_ref, out_ref):
    out_ref[...] = a_ref[...] + b_ref[...]


NBUF = 9
GDEPTH = 7


def _pick_chunk(per_w, d):
    budget = 440_000 // (NBUF * d * 4)
    c = (budget // 8) * 8
    while c % 8 != 0 or per_w % c != 0:
        c -= 8
    return c


def _make_row_gather(n_rows, n_out, d):
    per_w = n_out // NW
    chunk = _pick_chunk(per_w, d)
    n_chunks = per_w // chunk
    mesh = plsc.VectorSubcoreMesh(core_axis_name="c", subcore_axis_name="s")

    @functools.partial(
        pl.kernel, mesh=mesh,
        out_type=jax.ShapeDtypeStruct((n_out, d), jnp.float32),
        scratch_types=(
            [pltpu.VMEM((per_w,), jnp.int32)]
            + [pltpu.VMEM((chunk, d), jnp.float32)] * NBUF
            + [pltpu.SemaphoreType.DMA] * (2 * NBUF)
        ),
    )
    def gather(table_hbm, idx_hbm, out_hbm, idx_v, *bufs_sems):
        bufs = bufs_sems[:NBUF]
        gsems = bufs_sems[NBUF:2 * NBUF]
        wsems = bufs_sems[2 * NBUF:]
        wid = lax.axis_index("s") * NC + lax.axis_index("c")
        base = wid * per_w

        pltpu.sync_copy(idx_hbm.at[pl.ds(base, per_w)], idx_v)

        def fire_gather(c):
            b = c % NBUF
            return pltpu.async_copy(
                table_hbm.at[idx_v.at[pl.ds(c * chunk, chunk)]], bufs[b],
                gsems[b])

        gcp = [None] * NBUF
        wcp = [None] * NBUF
        for c in range(min(GDEPTH, n_chunks)):
            gcp[c % NBUF] = fire_gather(c)
        for c in range(n_chunks):
            b = c % NBUF
            gcp[b].wait()
            wcp[b] = pltpu.async_copy(
                bufs[b], out_hbm.at[pl.ds(base + c * chunk, chunk)], wsems[b])
            nxt = c + GDEPTH
            if nxt < n_chunks:
                nb = nxt % NBUF
                if wcp[nb] is not None:
                    wcp[nb].wait()
                gcp[nb] = fire_gather(nxt)
        for b in range(min(NBUF, n_chunks)):
            if wcp[b] is not None:
                wcp[b].wait()

    return gather


@jax.jit
def kernel(x, gate_w, w1, b1, w2, b2):
    bsz, seq, d = x.shape
    n = bsz * seq
    xf = x.reshape(n, d)
    two_f = w1.shape[1]
    n_assign = n * TOP_K
    n_groups = n_assign // GTILE + NUM_EXPERTS
    pad_len = n_groups * GTILE

    r = pl.pallas_call(
        _router_kernel,
        grid=(n // TILE,),
        in_specs=[
            pl.BlockSpec((TILE, d), lambda t: (t, 0)),
            pl.BlockSpec(gate_w.shape, lambda t: (0, 0)),
        ],
        out_specs=pl.BlockSpec((TILE, NUM_EXPERTS), lambda t: (t, 0)),
        out_shape=jax.ShapeDtypeStruct((n, NUM_EXPERTS), jnp.int32),
    )(xf, gate_w)

    e0 = jnp.argmax(r == 1, axis=1).astype(jnp.int32)
    e1 = jnp.argmax(r == 2, axis=1).astype(jnp.int32)
    expert_ids = jnp.concatenate([e0, e1])
    onehot = (expert_ids[:, None] == jnp.arange(NUM_EXPERTS)).astype(jnp.int32)
    incl = jnp.cumsum(onehot, axis=0)
    rank_within = jnp.sum((incl - onehot) * onehot, axis=1)
    counts = incl[-1]
    padded_counts = ((counts + GTILE - 1) // GTILE) * GTILE
    padded_offsets = jnp.concatenate(
        [jnp.zeros((1,), jnp.int32),
         jnp.cumsum(padded_counts)[:-1].astype(jnp.int32)])
    dest = padded_offsets[expert_ids] + rank_within
    tok = jnp.arange(n, dtype=jnp.int32)
    token_ids = jnp.concatenate([tok, tok])
    sorted_token = jnp.zeros((pad_len,), jnp.int32).at[dest].set(token_ids)
    tile_starts = jnp.arange(n_groups, dtype=jnp.int32) * GTILE
    tile_expert = jnp.clip(
        jnp.sum(tile_starts[:, None] >= padded_offsets[None, :], axis=1) - 1,
        0, NUM_EXPERTS - 1).astype(jnp.int32)

    xg = _make_row_gather(n, pad_len, d)(xf, sorted_token)

    yg = pl.pallas_call(
        _grouped_kernel,
        grid_spec=pltpu.PrefetchScalarGridSpec(
            num_scalar_prefetch=1,
            grid=(n_groups,),
            in_specs=[
                pl.BlockSpec((GTILE, d), lambda g, te: (g, 0)),
                pl.BlockSpec(gate_w.shape, lambda g, te: (0, 0)),
                pl.BlockSpec((1, two_f, d), lambda g, te: (te[g], 0, 0)),
                pl.BlockSpec((1, 1, two_f), lambda g, te: (te[g], 0, 0)),
                pl.BlockSpec((1, d, two_f // 2), lambda g, te: (te[g], 0, 0)),
                pl.BlockSpec((1, 1, d), lambda g, te: (te[g], 0, 0)),
            ],
            out_specs=pl.BlockSpec((GTILE, d), lambda g, te: (g, 0)),
        ),
        out_shape=jax.ShapeDtypeStruct((pad_len, d), jnp.float32),
    )(tile_expert, xg, gate_w, w1, b1.reshape(NUM_EXPERTS, 1, two_f), w2,
      b2.reshape(NUM_EXPERTS, 1, d))

    ygg = _make_row_gather(pad_len, n_assign, d)(yg, dest)

    nt = n // TILE
    out = pl.pallas_call(
        _add_kernel,
        grid=(nt,),
        in_specs=[
            pl.BlockSpec((TILE, d), lambda t: (t, 0)),
            pl.BlockSpec((TILE, d), lambda t: (t + nt, 0)),
        ],
        out_specs=pl.BlockSpec((TILE, d), lambda t: (t, 0)),
        out_shape=jax.ShapeDtypeStruct((n, d), jnp.float32),
    )(ygg, ygg)

    return out.reshape(bsz, seq, d), jnp.float32(0.0)


# speedup vs baseline: 1.6887x; 1.6784x over previous
import functools

import jax
import jax.numpy as jnp
from jax.experimental import pallas as pl

NUM_EXPERTS = 8
TOP_K = 2
TILE = 1024


def _moe_kernel(x_ref, gate_ref, w1_ref, b1_ref, w2_ref, b2_ref, out_ref):
    e = pl.program_id(0)
    t = pl.program_id(1)

    xt = x_ref[...]

    scores = jax.lax.dot_general(
        xt, gate_ref[...], (((1,), (1,)), ((), ())),
        preferred_element_type=jnp.float32)
    eidx = jax.lax.broadcasted_iota(jnp.int32, scores.shape, 1)
    m1 = jnp.max(scores, axis=-1, keepdims=True)
    top1 = jnp.min(jnp.where(scores == m1, eidx, NUM_EXPERTS),
                   axis=-1, keepdims=True)
    masked = jnp.where(eidx == top1, -jnp.inf, scores)
    m2 = jnp.max(masked, axis=-1, keepdims=True)
    top2 = jnp.min(jnp.where(masked == m2, eidx, NUM_EXPERTS),
                   axis=-1, keepdims=True)
    z2 = jnp.exp(m2 - m1)
    denom = 1.0 + z2
    p1 = 1.0 / denom
    p2 = z2 / denom
    weight = jnp.where(top1 == e, p1, 0.0) + jnp.where(top2 == e, p2, 0.0)

    w1e = w1_ref[0]
    h = jax.lax.dot_general(xt, w1e, (((1,), (1,)), ((), ())),
                            preferred_element_type=jnp.float32)
    h = h + b1_ref[0]
    f = h.shape[-1] // 2
    a = h[:, :f]
    g = h[:, f:]
    hidden = (a * jax.nn.sigmoid(a)) * g
    w2e = w2_ref[0]
    eo = jax.lax.dot_general(hidden, w2e, (((1,), (1,)), ((), ())),
                             preferred_element_type=jnp.float32)
    eo = (eo + b2_ref[0]) * weight

    rows = pl.ds(t * TILE, TILE)

    @pl.when(e == 0)
    def _init():
        out_ref[rows, :] = eo

    @pl.when(e != 0)
    def _acc():
        out_ref[rows, :] += eo


@functools.partial(jax.jit, static_argnames=())
def kernel(x, gate_w, w1, b1, w2, b2):
    bsz, seq, d = x.shape
    n = bsz * seq
    xf = x.reshape(n, d)
    two_f = w1.shape[1]
    n_tiles = n // TILE

    out = pl.pallas_call(
        _moe_kernel,
        grid=(NUM_EXPERTS, n_tiles),
        in_specs=[
            pl.BlockSpec((TILE, d), lambda e, t: (t, 0)),
            pl.BlockSpec(gate_w.shape, lambda e, t: (0, 0)),
            pl.BlockSpec((1, two_f, d), lambda e, t: (e, 0, 0)),
            pl.BlockSpec((1, 1, two_f), lambda e, t: (e, 0, 0)),
            pl.BlockSpec((1, d, two_f // 2), lambda e, t: (e, 0, 0)),
            pl.BlockSpec((1, 1, d), lambda e, t: (e, 0, 0)),
        ],
        out_specs=pl.BlockSpec((n, d), lambda e, t: (0, 0)),
        out_shape=jax.ShapeDtypeStruct((n, d), jnp.float32),
    )(xf, gate_w, w1, b1.reshape(NUM_EXPERTS, 1, two_f), w2,
      b2.reshape(NUM_EXPERTS, 1, d))

    return out.reshape(bsz, seq, d), jnp.float32(0.0)
